# baseline (speedup 1.0000x reference)
import jax
import jax.numpy as jnp
from jax.experimental import pallas as pl
from jax.experimental.pallas import tpu as pltpu

_N = 4096
_D = 2000
_H1 = 512
_H2 = 128
_HD = 512

_BM = 256


def _leaky(v):
    return jnp.where(v > 0, v, 0.01 * v)


def _dot(a, b):
    return jnp.dot(a, b, preferred_element_type=jnp.float32)


def _mm_kernel(x_ref, w_ref, o_ref):
    o_ref[...] = _dot(x_ref[...], w_ref[...])


def _gcn1_kernel(adj_ref, s1_ref, g2_ref, h1_ref, s2_ref):
    h1 = _leaky(_dot(adj_ref[...], s1_ref[...]))
    h1_ref[...] = h1
    s2_ref[...] = _dot(h1, g2_ref[...])


def _gcn2_kernel(adj_ref, s2_ref, fc1w_ref, fc1b_ref,
                 mu_ref, lv_ref, h_ref, stats_ref):
    i = pl.program_id(0)
    ml = _leaky(_dot(adj_ref[...], s2_ref[...]))
    mu = ml[:, :_H2]
    mu_ref[...] = mu
    lv_ref[...] = ml[:, _H2:]
    h = _dot(mu, fc1w_ref[...]) + fc1b_ref[...]
    h_ref[...] = h
    s = jnp.concatenate(
        [jnp.sum(h, axis=0, keepdims=True),
         jnp.sum(h * h, axis=0, keepdims=True)], axis=0)

    @pl.when(i == 0)
    def _init():
        stats_ref[...] = s

    @pl.when(i > 0)
    def _acc():
        stats_ref[...] += s


def _ip_kernel(zi_ref, zj_ref, o_ref):
    o_ref[...] = jax.lax.dot_general(
        zi_ref[...], zj_ref[...], (((1,), (1,)), ((), ())),
        preferred_element_type=jnp.float32)


def _head_kernel(h_ref, stats_ref, gamma_ref, beta_ref,
                 thw_ref, thb_ref, mw_ref, mb_ref, piw_ref, pib_ref,
                 out_ref, pi_ref, th_ref, mr_ref):
    s = stats_ref[...]
    bm = s[0:1, :] * (1.0 / _N)
    bv = s[1:2, :] * (1.0 / _N) - bm * bm
    inv = jax.lax.rsqrt(bv + 1e-5)
    out = _leaky((h_ref[...] - bm) * (inv * gamma_ref[...]) + beta_ref[...])
    out_ref[...] = out
    th = _dot(out, thw_ref[...]) + thb_ref[...]
    th_ref[...] = jnp.clip(jax.nn.softplus(th), 1e-5, 1e6)
    mn = _dot(out, mw_ref[...]) + mb_ref[...]
    mr_ref[...] = jnp.clip(jnp.exp(mn), 1e-5, 1e6)
    pi_ref[...] = jax.nn.sigmoid(mn * piw_ref[...] + pib_ref[...])


def kernel(x, adj, gc1_w, gc2_w, gc2s_w, fc1_w, fc1_b, fc1_gamma, fc1_beta,
           theta_w, theta_b, mean_w, mean_b, pi_w, pi_b):
    f32 = jnp.float32
    nblk = _N // _BM

    s1 = pl.pallas_call(
        _mm_kernel,
        grid=(nblk,),
        in_specs=[
            pl.BlockSpec((_BM, _D), lambda i: (i, 0)),
            pl.BlockSpec((_D, _H1), lambda i: (0, 0)),
        ],
        out_specs=pl.BlockSpec((_BM, _H1), lambda i: (i, 0)),
        out_shape=jax.ShapeDtypeStruct((_N, _H1), f32),
        compiler_params=pltpu.CompilerParams(
            dimension_semantics=("parallel",)),
    )(x, gc1_w)

    g2 = jnp.concatenate([gc2_w, gc2s_w], axis=1)
    h1, s2 = pl.pallas_call(
        _gcn1_kernel,
        grid=(nblk,),
        in_specs=[
            pl.BlockSpec((_BM, _N), lambda i: (i, 0)),
            pl.BlockSpec((_N, _H1), lambda i: (0, 0)),
            pl.BlockSpec((_H1, 2 * _H2), lambda i: (0, 0)),
        ],
        out_specs=[
            pl.BlockSpec((_BM, _H1), lambda i: (i, 0)),
            pl.BlockSpec((_BM, 2 * _H2), lambda i: (i, 0)),
        ],
        out_shape=[
            jax.ShapeDtypeStruct((_N, _H1), f32),
            jax.ShapeDtypeStruct((_N, 2 * _H2), f32),
        ],
        compiler_params=pltpu.CompilerParams(
            dimension_semantics=("parallel",)),
    )(adj, s1, g2)
    del h1

    fc1_b2 = fc1_b.reshape(1, _HD)
    mu, logvar, h, stats = pl.pallas_call(
        _gcn2_kernel,
        grid=(nblk,),
        in_specs=[
            pl.BlockSpec((_BM, _N), lambda i: (i, 0)),
            pl.BlockSpec((_N, 2 * _H2), lambda i: (0, 0)),
            pl.BlockSpec((_H2, _HD), lambda i: (0, 0)),
            pl.BlockSpec((1, _HD), lambda i: (0, 0)),
        ],
        out_specs=[
            pl.BlockSpec((_BM, _H2), lambda i: (i, 0)),
            pl.BlockSpec((_BM, _H2), lambda i: (i, 0)),
            pl.BlockSpec((_BM, _HD), lambda i: (i, 0)),
            pl.BlockSpec((2, _HD), lambda i: (0, 0)),
        ],
        out_shape=[
            jax.ShapeDtypeStruct((_N, _H2), f32),
            jax.ShapeDtypeStruct((_N, _H2), f32),
            jax.ShapeDtypeStruct((_N, _HD), f32),
            jax.ShapeDtypeStruct((2, _HD), f32),
        ],
        compiler_params=pltpu.CompilerParams(
            dimension_semantics=("arbitrary",)),
    )(adj, s2, fc1_w, fc1_b2)
    z = mu

    bz = 512
    adj_rec = pl.pallas_call(
        _ip_kernel,
        grid=(_N // bz, _N // bz),
        in_specs=[
            pl.BlockSpec((bz, _H2), lambda i, j: (i, 0)),
            pl.BlockSpec((bz, _H2), lambda i, j: (j, 0)),
        ],
        out_specs=pl.BlockSpec((bz, bz), lambda i, j: (i, j)),
        out_shape=jax.ShapeDtypeStruct((_N, _N), f32),
        compiler_params=pltpu.CompilerParams(
            dimension_semantics=("parallel", "parallel")),
    )(z, z)

    output, pi_res, theta_res, mean_res = pl.pallas_call(
        _head_kernel,
        grid=(nblk,),
        in_specs=[
            pl.BlockSpec((_BM, _HD), lambda i: (i, 0)),
            pl.BlockSpec((2, _HD), lambda i: (0, 0)),
            pl.BlockSpec((1, _HD), lambda i: (0, 0)),
            pl.BlockSpec((1, _HD), lambda i: (0, 0)),
            pl.BlockSpec((_HD, _D), lambda i: (0, 0)),
            pl.BlockSpec((1, _D), lambda i: (0, 0)),
            pl.BlockSpec((_HD, _D), lambda i: (0, 0)),
            pl.BlockSpec((1, _D), lambda i: (0, 0)),
            pl.BlockSpec((1, _D), lambda i: (0, 0)),
            pl.BlockSpec((1, _D), lambda i: (0, 0)),
        ],
        out_specs=[
            pl.BlockSpec((_BM, _HD), lambda i: (i, 0)),
            pl.BlockSpec((_BM, _D), lambda i: (i, 0)),
            pl.BlockSpec((_BM, _D), lambda i: (i, 0)),
            pl.BlockSpec((_BM, _D), lambda i: (i, 0)),
        ],
        out_shape=[
            jax.ShapeDtypeStruct((_N, _HD), f32),
            jax.ShapeDtypeStruct((_N, _D), f32),
            jax.ShapeDtypeStruct((_N, _D), f32),
            jax.ShapeDtypeStruct((_N, _D), f32),
        ],
        compiler_params=pltpu.CompilerParams(
            dimension_semantics=("parallel",)),
    )(h, stats, fc1_gamma.reshape(1, _HD), fc1_beta.reshape(1, _HD),
      theta_w, theta_b.reshape(1, _D), mean_w, mean_b.reshape(1, _D),
      pi_w.reshape(1, _D), pi_b.reshape(1, _D))

    return (adj_rec, mu, logvar, z, output, pi_res, theta_res, mean_res)


# speedup vs baseline: 1.0580x; 1.0580x over previous
import jax
import jax.numpy as jnp
from jax.experimental import pallas as pl
from jax.experimental.pallas import tpu as pltpu

_N = 4096
_D = 2000
_H1 = 512
_H2 = 128
_HD = 512

_BK = 256
_BI = 1024
_BM = 256


def _leaky(v):
    return jnp.where(v > 0, v, 0.01 * v)


def _dot(a, b):
    return jnp.dot(a, b, preferred_element_type=jnp.float32)


def _enc1_kernel(x_ref, adj_ref, gc1_ref, g2_ref, s2_hbm,
                 s1_scr, acc_scr, s2_scr, sem):
    t = pl.program_id(0)
    i = pl.program_id(1)
    ni = pl.num_programs(1)

    @pl.when(i == 0)
    def _make_s1():
        s1_scr[...] = _dot(x_ref[...], gc1_ref[...])

    part = _dot(adj_ref[...], s1_scr[...])
    isl = pl.ds(i * _BI, _BI)

    @pl.when(t == 0)
    def _init():
        acc_scr[isl, :] = part

    @pl.when(t > 0)
    def _acc():
        acc_scr[isl, :] += part

    @pl.when(t == _N // _BK - 1)
    def _final():
        s2_scr[isl, :] = _dot(_leaky(acc_scr[isl, :]), g2_ref[...])

        @pl.when(i == ni - 1)
        def _flush():
            cp = pltpu.make_async_copy(s2_scr, s2_hbm, sem)
            cp.start()
            cp.wait()


def _gcn2_kernel(adj_ref, s2_ref, fc1w_ref, fc1b_ref,
                 mu_ref, lv_ref, h_ref, stats_ref):
    i = pl.program_id(0)
    ml = _leaky(_dot(adj_ref[...], s2_ref[...]))
    mu = ml[:, :_H2]
    mu_ref[...] = mu
    lv_ref[...] = ml[:, _H2:]
    h = _dot(mu, fc1w_ref[...]) + fc1b_ref[...]
    h_ref[...] = h
    s = jnp.concatenate(
        [jnp.sum(h, axis=0, keepdims=True),
         jnp.sum(h * h, axis=0, keepdims=True)], axis=0)

    @pl.when(i == 0)
    def _init():
        stats_ref[...] = s

    @pl.when(i > 0)
    def _acc():
        stats_ref[...] += s


def _dec_kernel(z_ref, zi_ref, h_ref, stats_ref, gamma_ref, beta_ref,
                thw_ref, thb_ref, mw_ref, mb_ref, piw_ref, pib_ref,
                rec_ref, out_ref, pi_ref, th_ref, mr_ref):
    rec_ref[...] = jax.lax.dot_general(
        zi_ref[...], z_ref[...], (((1,), (1,)), ((), ())),
        preferred_element_type=jnp.float32)
    s = stats_ref[...]
    bm = s[0:1, :] * (1.0 / _N)
    bv = s[1:2, :] * (1.0 / _N) - bm * bm
    inv = jax.lax.rsqrt(bv + 1e-5)
    out = _leaky((h_ref[...] - bm) * (inv * gamma_ref[...]) + beta_ref[...])
    out_ref[...] = out
    th = _dot(out, thw_ref[...]) + thb_ref[...]
    th_ref[...] = jnp.clip(jax.nn.softplus(th), 1e-5, 1e6)
    mn = _dot(out, mw_ref[...]) + mb_ref[...]
    mr_ref[...] = jnp.clip(jnp.exp(mn), 1e-5, 1e6)
    pi_ref[...] = jax.nn.sigmoid(mn * piw_ref[...] + pib_ref[...])


def kernel(x, adj, gc1_w, gc2_w, gc2s_w, fc1_w, fc1_b, fc1_gamma, fc1_beta,
           theta_w, theta_b, mean_w, mean_b, pi_w, pi_b):
    f32 = jnp.float32

    g2 = jnp.concatenate([gc2_w, gc2s_w], axis=1)
    s2 = pl.pallas_call(
        _enc1_kernel,
        grid=(_N // _BK, _N // _BI),
        in_specs=[
            pl.BlockSpec((_BK, _D), lambda t, i: (t, 0)),
            pl.BlockSpec((_BI, _BK), lambda t, i: (i, t)),
            pl.BlockSpec((_D, _H1), lambda t, i: (0, 0)),
            pl.BlockSpec((_H1, 2 * _H2), lambda t, i: (0, 0)),
        ],
        out_specs=pl.BlockSpec(memory_space=pltpu.MemorySpace.HBM),
        out_shape=jax.ShapeDtypeStruct((_N, 2 * _H2), f32),
        scratch_shapes=[
            pltpu.VMEM((_BK, _H1), f32),
            pltpu.VMEM((_N, _H1), f32),
            pltpu.VMEM((_N, 2 * _H2), f32),
            pltpu.SemaphoreType.DMA,
        ],
        compiler_params=pltpu.CompilerParams(
            dimension_semantics=("arbitrary", "arbitrary")),
    )(x, adj, gc1_w, g2)

    nblk = _N // _BM
    fc1_b2 = fc1_b.reshape(1, _HD)
    mu, logvar, h, stats = pl.pallas_call(
        _gcn2_kernel,
        grid=(nblk,),
        in_specs=[
            pl.BlockSpec((_BM, _N), lambda i: (i, 0)),
            pl.BlockSpec((_N, 2 * _H2), lambda i: (0, 0)),
            pl.BlockSpec((_H2, _HD), lambda i: (0, 0)),
            pl.BlockSpec((1, _HD), lambda i: (0, 0)),
        ],
        out_specs=[
            pl.BlockSpec((_BM, _H2), lambda i: (i, 0)),
            pl.BlockSpec((_BM, _H2), lambda i: (i, 0)),
            pl.BlockSpec((_BM, _HD), lambda i: (i, 0)),
            pl.BlockSpec((2, _HD), lambda i: (0, 0)),
        ],
        out_shape=[
            jax.ShapeDtypeStruct((_N, _H2), f32),
            jax.ShapeDtypeStruct((_N, _H2), f32),
            jax.ShapeDtypeStruct((_N, _HD), f32),
            jax.ShapeDtypeStruct((2, _HD), f32),
        ],
        compiler_params=pltpu.CompilerParams(
            dimension_semantics=("arbitrary",)),
    )(adj, s2, fc1_w, fc1_b2)
    z = mu

    adj_rec, output, pi_res, theta_res, mean_res = pl.pallas_call(
        _dec_kernel,
        grid=(nblk,),
        in_specs=[
            pl.BlockSpec((_N, _H2), lambda i: (0, 0)),
            pl.BlockSpec((_BM, _H2), lambda i: (i, 0)),
            pl.BlockSpec((_BM, _HD), lambda i: (i, 0)),
            pl.BlockSpec((2, _HD), lambda i: (0, 0)),
            pl.BlockSpec((1, _HD), lambda i: (0, 0)),
            pl.BlockSpec((1, _HD), lambda i: (0, 0)),
            pl.BlockSpec((_HD, _D), lambda i: (0, 0)),
            pl.BlockSpec((1, _D), lambda i: (0, 0)),
            pl.BlockSpec((_HD, _D), lambda i: (0, 0)),
            pl.BlockSpec((1, _D), lambda i: (0, 0)),
            pl.BlockSpec((1, _D), lambda i: (0, 0)),
            pl.BlockSpec((1, _D), lambda i: (0, 0)),
        ],
        out_specs=[
            pl.BlockSpec((_BM, _N), lambda i: (i, 0)),
            pl.BlockSpec((_BM, _HD), lambda i: (i, 0)),
            pl.BlockSpec((_BM, _D), lambda i: (i, 0)),
            pl.BlockSpec((_BM, _D), lambda i: (i, 0)),
            pl.BlockSpec((_BM, _D), lambda i: (i, 0)),
        ],
        out_shape=[
            jax.ShapeDtypeStruct((_N, _N), f32),
            jax.ShapeDtypeStruct((_N, _HD), f32),
            jax.ShapeDtypeStruct((_N, _D), f32),
            jax.ShapeDtypeStruct((_N, _D), f32),
            jax.ShapeDtypeStruct((_N, _D), f32),
        ],
        compiler_params=pltpu.CompilerParams(
            dimension_semantics=("arbitrary",)),
    )(z, z, h, stats, fc1_gamma.reshape(1, _HD), fc1_beta.reshape(1, _HD),
      theta_w, theta_b.reshape(1, _D), mean_w, mean_b.reshape(1, _D),
      pi_w.reshape(1, _D), pi_b.reshape(1, _D))

    return (adj_rec, mu, logvar, z, output, pi_res, theta_res, mean_res)


# speedup vs baseline: 1.1984x; 1.1327x over previous
import jax
import jax.numpy as jnp
from jax.experimental import pallas as pl
from jax.experimental.pallas import tpu as pltpu

_N = 4096
_D = 2000
_H1 = 512
_H2 = 128
_HD = 512

_BK = 256
_BI = 1024
_BM = 256


def _leaky(v):
    return jnp.where(v > 0, v, 0.01 * v)


def _dot(a, b):
    return jnp.dot(a, b, preferred_element_type=jnp.float32)


def _enc1_kernel(x_ref, adj_ref, gc1_ref, g2_ref, s2_ref, s1_scr):
    t = pl.program_id(0)
    nk = _N // _BK

    @pl.when(t < nk)
    def _phase1():
        s1_scr[pl.ds(t * _BK, _BK), :] = _dot(x_ref[...], gc1_ref[...])

    @pl.when(t >= nk)
    def _phase2():
        h1 = _leaky(_dot(adj_ref[...], s1_scr[...]))
        s2_ref[...] = _dot(h1, g2_ref[...])


def _gcn2_kernel(adj_ref, s2_ref, fc1w_ref, fc1b_ref,
                 mu_ref, lv_ref, h_ref, stats_ref):
    i = pl.program_id(0)
    ml = _leaky(_dot(adj_ref[...], s2_ref[...]))
    mu = ml[:, :_H2]
    mu_ref[...] = mu
    lv_ref[...] = ml[:, _H2:]
    h = _dot(mu, fc1w_ref[...]) + fc1b_ref[...]
    h_ref[...] = h
    s = jnp.concatenate(
        [jnp.sum(h, axis=0, keepdims=True),
         jnp.sum(h * h, axis=0, keepdims=True)], axis=0)

    @pl.when(i == 0)
    def _init():
        stats_ref[...] = s

    @pl.when(i > 0)
    def _acc():
        stats_ref[...] += s


def _dec_kernel(z_ref, zi_ref, h_ref, stats_ref, gamma_ref, beta_ref,
                thw_ref, thb_ref, mw_ref, mb_ref, piw_ref, pib_ref,
                rec_ref, out_ref, pi_ref, th_ref, mr_ref):
    rec_ref[...] = jax.lax.dot_general(
        zi_ref[...], z_ref[...], (((1,), (1,)), ((), ())),
        preferred_element_type=jnp.float32)
    s = stats_ref[...]
    bm = s[0:1, :] * (1.0 / _N)
    bv = s[1:2, :] * (1.0 / _N) - bm * bm
    inv = jax.lax.rsqrt(bv + 1e-5)
    out = _leaky((h_ref[...] - bm) * (inv * gamma_ref[...]) + beta_ref[...])
    out_ref[...] = out
    th = _dot(out, thw_ref[...]) + thb_ref[...]
    th_ref[...] = jnp.clip(jax.nn.softplus(th), 1e-5, 1e6)
    mn = _dot(out, mw_ref[...]) + mb_ref[...]
    mr_ref[...] = jnp.clip(jnp.exp(mn), 1e-5, 1e6)
    pi_ref[...] = jax.nn.sigmoid(mn * piw_ref[...] + pib_ref[...])


def kernel(x, adj, gc1_w, gc2_w, gc2s_w, fc1_w, fc1_b, fc1_gamma, fc1_beta,
           theta_w, theta_b, mean_w, mean_b, pi_w, pi_b):
    f32 = jnp.float32

    g2 = jnp.concatenate([gc2_w, gc2s_w], axis=1)
    nk = _N // _BK
    s2 = pl.pallas_call(
        _enc1_kernel,
        grid=(2 * nk,),
        in_specs=[
            pl.BlockSpec((_BK, _D), lambda t: (jnp.minimum(t, nk - 1), 0)),
            pl.BlockSpec((_BK, _N),
                         lambda t: (jnp.maximum(t - nk, 0), 0)),
            pl.BlockSpec((_D, _H1), lambda t: (0, 0)),
            pl.BlockSpec((_H1, 2 * _H2), lambda t: (0, 0)),
        ],
        out_specs=pl.BlockSpec((_BK, 2 * _H2),
                               lambda t: (jnp.maximum(t - nk, 0), 0)),
        out_shape=jax.ShapeDtypeStruct((_N, 2 * _H2), f32),
        scratch_shapes=[
            pltpu.VMEM((_N, _H1), f32),
        ],
        compiler_params=pltpu.CompilerParams(
            dimension_semantics=("arbitrary",)),
    )(x, adj, gc1_w, g2)

    nblk = _N // _BM
    fc1_b2 = fc1_b.reshape(1, _HD)
    mu, logvar, h, stats = pl.pallas_call(
        _gcn2_kernel,
        grid=(nblk,),
        in_specs=[
            pl.BlockSpec((_BM, _N), lambda i: (i, 0)),
            pl.BlockSpec((_N, 2 * _H2), lambda i: (0, 0)),
            pl.BlockSpec((_H2, _HD), lambda i: (0, 0)),
            pl.BlockSpec((1, _HD), lambda i: (0, 0)),
        ],
        out_specs=[
            pl.BlockSpec((_BM, _H2), lambda i: (i, 0)),
            pl.BlockSpec((_BM, _H2), lambda i: (i, 0)),
            pl.BlockSpec((_BM, _HD), lambda i: (i, 0)),
            pl.BlockSpec((2, _HD), lambda i: (0, 0)),
        ],
        out_shape=[
            jax.ShapeDtypeStruct((_N, _H2), f32),
            jax.ShapeDtypeStruct((_N, _H2), f32),
            jax.ShapeDtypeStruct((_N, _HD), f32),
            jax.ShapeDtypeStruct((2, _HD), f32),
        ],
        compiler_params=pltpu.CompilerParams(
            dimension_semantics=("arbitrary",)),
    )(adj, s2, fc1_w, fc1_b2)
    z = mu

    adj_rec, output, pi_res, theta_res, mean_res = pl.pallas_call(
        _dec_kernel,
        grid=(nblk,),
        in_specs=[
            pl.BlockSpec((_N, _H2), lambda i: (0, 0)),
            pl.BlockSpec((_BM, _H2), lambda i: (i, 0)),
            pl.BlockSpec((_BM, _HD), lambda i: (i, 0)),
            pl.BlockSpec((2, _HD), lambda i: (0, 0)),
            pl.BlockSpec((1, _HD), lambda i: (0, 0)),
            pl.BlockSpec((1, _HD), lambda i: (0, 0)),
            pl.BlockSpec((_HD, _D), lambda i: (0, 0)),
            pl.BlockSpec((1, _D), lambda i: (0, 0)),
            pl.BlockSpec((_HD, _D), lambda i: (0, 0)),
            pl.BlockSpec((1, _D), lambda i: (0, 0)),
            pl.BlockSpec((1, _D), lambda i: (0, 0)),
            pl.BlockSpec((1, _D), lambda i: (0, 0)),
        ],
        out_specs=[
            pl.BlockSpec((_BM, _N), lambda i: (i, 0)),
            pl.BlockSpec((_BM, _HD), lambda i: (i, 0)),
            pl.BlockSpec((_BM, _D), lambda i: (i, 0)),
            pl.BlockSpec((_BM, _D), lambda i: (i, 0)),
            pl.BlockSpec((_BM, _D), lambda i: (i, 0)),
        ],
        out_shape=[
            jax.ShapeDtypeStruct((_N, _N), f32),
            jax.ShapeDtypeStruct((_N, _HD), f32),
            jax.ShapeDtypeStruct((_N, _D), f32),
            jax.ShapeDtypeStruct((_N, _D), f32),
            jax.ShapeDtypeStruct((_N, _D), f32),
        ],
        compiler_params=pltpu.CompilerParams(
            dimension_semantics=("arbitrary",)),
    )(z, z, h, stats, fc1_gamma.reshape(1, _HD), fc1_beta.reshape(1, _HD),
      theta_w, theta_b.reshape(1, _D), mean_w, mean_b.reshape(1, _D),
      pi_w.reshape(1, _D), pi_b.reshape(1, _D))

    return (adj_rec, mu, logvar, z, output, pi_res, theta_res, mean_res)


# speedup vs baseline: 1.2175x; 1.0160x over previous
import jax
import jax.numpy as jnp
from jax.experimental import pallas as pl
from jax.experimental.pallas import tpu as pltpu

_N = 4096
_D = 2000
_H1 = 512
_H2 = 128
_HD = 512

_BK = 256
_BI = 1024
_BM = 512


def _leaky(v):
    return jnp.where(v > 0, v, 0.01 * v)


def _dot(a, b):
    return jnp.dot(a, b, preferred_element_type=jnp.float32)


def _enc1_kernel(x_ref, adj_ref, gc1_ref, g2_ref, s2_ref, s1_scr):
    t = pl.program_id(0)
    nk = _N // _BK

    @pl.when(t < nk)
    def _phase1():
        s1_scr[pl.ds(t * _BK, _BK), :] = _dot(x_ref[...], gc1_ref[...])

    @pl.when(t >= nk)
    def _phase2():
        h1 = _leaky(_dot(adj_ref[...], s1_scr[...]))
        s2_ref[...] = _dot(h1, g2_ref[...])


def _gcn2_kernel(adj_ref, s2_ref, fc1w_ref, fc1b_ref,
                 mu_ref, lv_ref, h_ref, stats_ref):
    i = pl.program_id(0)
    ml = _leaky(_dot(adj_ref[...], s2_ref[...]))
    mu = ml[:, :_H2]
    mu_ref[...] = mu
    lv_ref[...] = ml[:, _H2:]
    h = _dot(mu, fc1w_ref[...]) + fc1b_ref[...]
    h_ref[...] = h
    s = jnp.concatenate(
        [jnp.sum(h, axis=0, keepdims=True),
         jnp.sum(h * h, axis=0, keepdims=True)], axis=0)

    @pl.when(i == 0)
    def _init():
        stats_ref[...] = s

    @pl.when(i > 0)
    def _acc():
        stats_ref[...] += s


def _dec_kernel(z_ref, zi_ref, h_ref, stats_ref, gamma_ref, beta_ref,
                thw_ref, thb_ref, mw_ref, mb_ref, piw_ref, pib_ref,
                rec_ref, out_ref, pi_ref, th_ref, mr_ref):
    rec_ref[...] = jax.lax.dot_general(
        zi_ref[...], z_ref[...], (((1,), (1,)), ((), ())),
        preferred_element_type=jnp.float32)
    s = stats_ref[...]
    bm = s[0:1, :] * (1.0 / _N)
    bv = s[1:2, :] * (1.0 / _N) - bm * bm
    inv = jax.lax.rsqrt(bv + 1e-5)
    out = _leaky((h_ref[...] - bm) * (inv * gamma_ref[...]) + beta_ref[...])
    out_ref[...] = out
    th = _dot(out, thw_ref[...]) + thb_ref[...]
    th_ref[...] = jnp.clip(jax.nn.softplus(th), 1e-5, 1e6)
    mn = _dot(out, mw_ref[...]) + mb_ref[...]
    mr_ref[...] = jnp.clip(jnp.exp(mn), 1e-5, 1e6)
    pi_ref[...] = jax.nn.sigmoid(mn * piw_ref[...] + pib_ref[...])


def kernel(x, adj, gc1_w, gc2_w, gc2s_w, fc1_w, fc1_b, fc1_gamma, fc1_beta,
           theta_w, theta_b, mean_w, mean_b, pi_w, pi_b):
    f32 = jnp.float32

    g2 = jnp.concatenate([gc2_w, gc2s_w], axis=1)
    nk = _N // _BK
    s2 = pl.pallas_call(
        _enc1_kernel,
        grid=(2 * nk,),
        in_specs=[
            pl.BlockSpec((_BK, _D), lambda t: (jnp.minimum(t, nk - 1), 0)),
            pl.BlockSpec((_BK, _N),
                         lambda t: (jnp.maximum(t - nk, 0), 0)),
            pl.BlockSpec((_D, _H1), lambda t: (0, 0)),
            pl.BlockSpec((_H1, 2 * _H2), lambda t: (0, 0)),
        ],
        out_specs=pl.BlockSpec((_BK, 2 * _H2),
                               lambda t: (jnp.maximum(t - nk, 0), 0)),
        out_shape=jax.ShapeDtypeStruct((_N, 2 * _H2), f32),
        scratch_shapes=[
            pltpu.VMEM((_N, _H1), f32),
        ],
        compiler_params=pltpu.CompilerParams(
            dimension_semantics=("arbitrary",)),
    )(x, adj, gc1_w, g2)

    nblk = _N // _BM
    fc1_b2 = fc1_b.reshape(1, _HD)
    mu, logvar, h, stats = pl.pallas_call(
        _gcn2_kernel,
        grid=(nblk,),
        in_specs=[
            pl.BlockSpec((_BM, _N), lambda i: (i, 0)),
            pl.BlockSpec((_N, 2 * _H2), lambda i: (0, 0)),
            pl.BlockSpec((_H2, _HD), lambda i: (0, 0)),
            pl.BlockSpec((1, _HD), lambda i: (0, 0)),
        ],
        out_specs=[
            pl.BlockSpec((_BM, _H2), lambda i: (i, 0)),
            pl.BlockSpec((_BM, _H2), lambda i: (i, 0)),
            pl.BlockSpec((_BM, _HD), lambda i: (i, 0)),
            pl.BlockSpec((2, _HD), lambda i: (0, 0)),
        ],
        out_shape=[
            jax.ShapeDtypeStruct((_N, _H2), f32),
            jax.ShapeDtypeStruct((_N, _H2), f32),
            jax.ShapeDtypeStruct((_N, _HD), f32),
            jax.ShapeDtypeStruct((2, _HD), f32),
        ],
        compiler_params=pltpu.CompilerParams(
            dimension_semantics=("arbitrary",)),
    )(adj, s2, fc1_w, fc1_b2)
    z = mu

    adj_rec, output, pi_res, theta_res, mean_res = pl.pallas_call(
        _dec_kernel,
        grid=(nblk,),
        in_specs=[
            pl.BlockSpec((_N, _H2), lambda i: (0, 0)),
            pl.BlockSpec((_BM, _H2), lambda i: (i, 0)),
            pl.BlockSpec((_BM, _HD), lambda i: (i, 0)),
            pl.BlockSpec((2, _HD), lambda i: (0, 0)),
            pl.BlockSpec((1, _HD), lambda i: (0, 0)),
            pl.BlockSpec((1, _HD), lambda i: (0, 0)),
            pl.BlockSpec((_HD, _D), lambda i: (0, 0)),
            pl.BlockSpec((1, _D), lambda i: (0, 0)),
            pl.BlockSpec((_HD, _D), lambda i: (0, 0)),
            pl.BlockSpec((1, _D), lambda i: (0, 0)),
            pl.BlockSpec((1, _D), lambda i: (0, 0)),
            pl.BlockSpec((1, _D), lambda i: (0, 0)),
        ],
        out_specs=[
            pl.BlockSpec((_BM, _N), lambda i: (i, 0)),
            pl.BlockSpec((_BM, _HD), lambda i: (i, 0)),
            pl.BlockSpec((_BM, _D), lambda i: (i, 0)),
            pl.BlockSpec((_BM, _D), lambda i: (i, 0)),
            pl.BlockSpec((_BM, _D), lambda i: (i, 0)),
        ],
        out_shape=[
            jax.ShapeDtypeStruct((_N, _N), f32),
            jax.ShapeDtypeStruct((_N, _HD), f32),
            jax.ShapeDtypeStruct((_N, _D), f32),
            jax.ShapeDtypeStruct((_N, _D), f32),
            jax.ShapeDtypeStruct((_N, _D), f32),
        ],
        compiler_params=pltpu.CompilerParams(
            dimension_semantics=("arbitrary",)),
    )(z, z, h, stats, fc1_gamma.reshape(1, _HD), fc1_beta.reshape(1, _HD),
      theta_w, theta_b.reshape(1, _D), mean_w, mean_b.reshape(1, _D),
      pi_w.reshape(1, _D), pi_b.reshape(1, _D))

    return (adj_rec, mu, logvar, z, output, pi_res, theta_res, mean_res)
